# baseline (speedup 1.0000x reference)
import functools

import jax
import jax.numpy as jnp
from jax import lax
from jax.experimental import pallas as pl
from jax.experimental.pallas import tpu as pltpu
from jax.experimental.pallas import tpu_sc as plsc

N = 10000
E = 320000
D = 128
H = 8
F = 16
L = 16
DM = D + L
NC = 2
NS = 16
NW = NC * NS
EPW = E // NW
K = 80
NCHUNK = EPW // K
NBLK = N // K

BN = 1000


def _tc_pre_body(x_ref, w_ref, ab_ref, ba_ref, h_ref, elr_ref, erl_ref):
    h = jnp.dot(x_ref[...], w_ref[...], preferred_element_type=jnp.float32)
    h_ref[...] = jnp.concatenate([h, jnp.zeros((BN, L), jnp.float32)], axis=1)
    elr_ref[...] = jnp.dot(h, ab_ref[...], preferred_element_type=jnp.float32)
    erl_ref[...] = jnp.dot(h, ba_ref[...], preferred_element_type=jnp.float32)


def _tc_pre(x, W, AB, BA):
    return pl.pallas_call(
        _tc_pre_body,
        grid=(N // BN,),
        in_specs=[
            pl.BlockSpec((BN, D), lambda i: (i, 0)),
            pl.BlockSpec((D, D), lambda i: (0, 0)),
            pl.BlockSpec((D, 2 * H), lambda i: (0, 0)),
            pl.BlockSpec((D, 2 * H), lambda i: (0, 0)),
        ],
        out_specs=[
            pl.BlockSpec((BN, DM), lambda i: (i, 0)),
            pl.BlockSpec((BN, 2 * H), lambda i: (i, 0)),
            pl.BlockSpec((BN, 2 * H), lambda i: (i, 0)),
        ],
        out_shape=[
            jax.ShapeDtypeStruct((N, DM), jnp.float32),
            jax.ShapeDtypeStruct((N, 2 * H), jnp.float32),
            jax.ShapeDtypeStruct((N, 2 * H), jnp.float32),
        ],
    )(x, W, AB, BA)


def _norm_block(a0_ref, a1_ref, b_ref, r_ref):
    a = a0_ref[...] + a1_ref[...]
    feat = a[:, :D]
    s8 = a[:, D:D + H]
    rec = jnp.where(s8 > 0, 1.0 / s8, 0.0)
    rbig = jnp.dot(rec, r_ref[...], preferred_element_type=jnp.float32)
    return feat * rbig + b_ref[...]


def _tc_post1_body(a0_ref, a1_ref, b_ref, r_ref, w2_ref, ab2_ref, ba2_ref,
                   h2_ref, elr2_ref, erl2_ref):
    o = _norm_block(a0_ref, a1_ref, b_ref, r_ref)
    o = jnp.where(o > 0, o, jnp.exp(o) - 1.0)
    h2 = jnp.dot(o, w2_ref[...], preferred_element_type=jnp.float32)
    h2_ref[...] = jnp.concatenate([h2, jnp.zeros((BN, L), jnp.float32)], axis=1)
    elr2_ref[...] = jnp.dot(h2, ab2_ref[...], preferred_element_type=jnp.float32)
    erl2_ref[...] = jnp.dot(h2, ba2_ref[...], preferred_element_type=jnp.float32)


def _tc_post1(a0, a1, b, R, W2, AB2, BA2):
    return pl.pallas_call(
        _tc_post1_body,
        grid=(N // BN,),
        in_specs=[
            pl.BlockSpec((BN, DM), lambda i: (i, 0)),
            pl.BlockSpec((BN, DM), lambda i: (i, 0)),
            pl.BlockSpec((1, D), lambda i: (0, 0)),
            pl.BlockSpec((H, D), lambda i: (0, 0)),
            pl.BlockSpec((D, D), lambda i: (0, 0)),
            pl.BlockSpec((D, 2 * H), lambda i: (0, 0)),
            pl.BlockSpec((D, 2 * H), lambda i: (0, 0)),
        ],
        out_specs=[
            pl.BlockSpec((BN, DM), lambda i: (i, 0)),
            pl.BlockSpec((BN, 2 * H), lambda i: (i, 0)),
            pl.BlockSpec((BN, 2 * H), lambda i: (i, 0)),
        ],
        out_shape=[
            jax.ShapeDtypeStruct((N, DM), jnp.float32),
            jax.ShapeDtypeStruct((N, 2 * H), jnp.float32),
            jax.ShapeDtypeStruct((N, 2 * H), jnp.float32),
        ],
    )(a0, a1, b, R, W2, AB2, BA2)


def _tc_post2_body(a0_ref, a1_ref, b_ref, r_ref, m_ref, out_ref):
    o = _norm_block(a0_ref, a1_ref, b_ref, r_ref)
    out_ref[...] = jnp.dot(o, m_ref[...], preferred_element_type=jnp.float32)


def _tc_post2(a0, a1, b, R, M):
    return pl.pallas_call(
        _tc_post2_body,
        grid=(N // BN,),
        in_specs=[
            pl.BlockSpec((BN, DM), lambda i: (i, 0)),
            pl.BlockSpec((BN, DM), lambda i: (i, 0)),
            pl.BlockSpec((1, D), lambda i: (0, 0)),
            pl.BlockSpec((H, D), lambda i: (0, 0)),
            pl.BlockSpec((D, F), lambda i: (0, 0)),
        ],
        out_specs=pl.BlockSpec((BN, F), lambda i: (i, 0)),
        out_shape=jax.ShapeDtypeStruct((N, F), jnp.float32),
    )(a0, a1, b, R, M)


def _sc_edge_body(h_hbm, elr_hbm, erl_hbm, src_hbm, dst_hbm, acc_out,
                  acc_sh, srcv, dstv, dsts, hrows, elsrc, erdst,
                  gsem, ssem, isem):
    c = lax.axis_index("c")
    s_ = lax.axis_index("s")
    zero16 = jnp.zeros((L,), jnp.float32)
    lane_iota = lax.iota(jnp.int32, L)

    def _zero_buf(i, _):
        for jj in range(DM // L):
            hrows[0, i, pl.ds(jj * L, L)] = zero16
        return 0
    lax.fori_loop(0, K, _zero_buf, 0)

    def _zero_acc(b, _):
        @pl.when((b % NS) == s_)
        def _():
            pltpu.sync_copy(hrows.at[0],
                            acc_sh.at[pl.ds(pl.multiple_of(b * K, 8), K)])
        return 0
    lax.fori_loop(0, NBLK, _zero_acc, 0)

    plsc.subcore_barrier()

    ebase = (c * NS + s_) * EPW

    def _fire_idx(j, par):
        base = pl.multiple_of(ebase + j * K, 8)
        pltpu.async_copy(src_hbm.at[pl.ds(base, K)], srcv.at[par], isem)
        pltpu.async_copy(dst_hbm.at[pl.ds(base, K)], dstv.at[par], isem)

    def _wait_idx(par):
        pltpu.make_async_copy(src_hbm.at[pl.ds(0, K)], srcv.at[par], isem).wait()
        pltpu.make_async_copy(dst_hbm.at[pl.ds(0, K)], dstv.at[par], isem).wait()

    def _fire_gathers(ring, par):
        pltpu.async_copy(h_hbm.at[srcv.at[par]], hrows.at[ring], gsem)
        pltpu.async_copy(elr_hbm.at[srcv.at[par]], elsrc.at[par], gsem)
        pltpu.async_copy(erl_hbm.at[dstv.at[par]], erdst.at[par], gsem)

    def _drain_gathers():
        pltpu.make_async_copy(h_hbm.at[srcv.at[0]], hrows.at[0], gsem).wait()
        pltpu.make_async_copy(elr_hbm.at[srcv.at[0]], elsrc.at[0], gsem).wait()
        pltpu.make_async_copy(erl_hbm.at[dstv.at[0]], erdst.at[0], gsem).wait()

    K2 = K // 2

    def _drain_scatter():
        pltpu.make_async_copy(hrows.at[0, pl.ds(0, K2)],
                              acc_sh.at[dsts.at[0, 0]], ssem).wait()
        pltpu.make_async_copy(hrows.at[0, pl.ds(0, K2)],
                              acc_sh.at[dsts.at[0, 1]], ssem).wait()

    def _compute(ring, par):
        def _edge(kk, _):
            for u in range(4):
                k = kk * 4 + u
                e16 = elsrc[par, k, :] + erdst[par, k, :]
                e16 = jnp.where(e16 > 0.0, e16, 0.2 * e16)
                ee = jnp.where(lane_iota < H, jnp.exp(e16), 0.0)
                hrows[ring, k, pl.ds(D, L)] = ee
                for hh in range(H):
                    hrows[ring, k, pl.ds(hh * L, L)] = (
                        hrows[ring, k, pl.ds(hh * L, L)] * ee[hh])
            return 0
        lax.fori_loop(0, K // 4, _edge, 0)

    def _fire_scatter(ring, par):
        for half in range(2):
            for off in (0, L, K2 - L):
                dsts[par, half, pl.ds(off, L)] = (
                    dstv[par, pl.ds(half * K2 + off, L)])
            pltpu.async_copy(hrows.at[ring, pl.ds(half * K2, K2)],
                             acc_sh.at[dsts.at[par, half]], ssem, add=True)

    pltpu.sync_copy(src_hbm.at[pl.ds(pl.multiple_of(ebase, 8), K)], srcv.at[0])
    pltpu.sync_copy(dst_hbm.at[pl.ds(pl.multiple_of(ebase, 8), K)], dstv.at[0])
    _fire_gathers(0, 0)
    _fire_idx(1, 1)

    def _iter(j, _):
        p2 = lax.rem(j, 2)
        p3 = lax.rem(j, 3)
        _drain_gathers()

        @pl.when(j >= 2)
        def _():
            _drain_scatter()

        @pl.when(j <= NCHUNK - 2)
        def _():
            _wait_idx(1 - p2)
            _fire_gathers(lax.rem(j + 1, 3), 1 - p2)

        _compute(p3, p2)
        _fire_scatter(p3, p2)

        @pl.when(j <= NCHUNK - 3)
        def _():
            _fire_idx(j + 2, p2)
        return 0

    lax.fori_loop(0, NCHUNK, _iter, 0)
    _drain_scatter()
    _drain_scatter()

    plsc.subcore_barrier()

    def _wb(b, _):
        @pl.when((b % NS) == s_)
        def _():
            r = pl.multiple_of(b * K, 8)
            pltpu.sync_copy(acc_sh.at[pl.ds(r, K)], hrows.at[0])
            pltpu.sync_copy(hrows.at[0], acc_out.at[c, pl.ds(r, K)])
        return 0
    lax.fori_loop(0, NBLK, _wb, 0)


@functools.lru_cache(maxsize=1)
def _sc_edges_fn():
    return pl.kernel(
        _sc_edge_body,
        out_type=jax.ShapeDtypeStruct((NC, N, DM), jnp.float32),
        mesh=plsc.VectorSubcoreMesh(core_axis_name="c", subcore_axis_name="s",
                                    num_cores=NC, num_subcores=NS),
        compiler_params=pltpu.CompilerParams(use_tc_tiling_on_sc=False),
        scratch_types=[
            pltpu.VMEM_SHARED((N, DM), jnp.float32),
            pltpu.VMEM((2, K), jnp.int32),
            pltpu.VMEM((2, K), jnp.int32),
            pltpu.VMEM((2, 2, K // 2), jnp.int32),
            pltpu.VMEM((3, K, DM), jnp.float32),
            pltpu.VMEM((2, K, L), jnp.float32),
            pltpu.VMEM((2, K, L), jnp.float32),
            pltpu.SemaphoreType.DMA,
            pltpu.SemaphoreType.DMA,
            pltpu.SemaphoreType.DMA,
        ],
    )


def _sc_edges(h, elr, erl, src, dst):
    return _sc_edges_fn()(h, elr, erl, src, dst)


def _attn_mat(al, ar):
    eye = jnp.eye(H, dtype=jnp.float32)
    A = (eye[:, None, :] * al[:, :, None]).reshape(D, H)
    B = (eye[:, None, :] * ar[:, :, None]).reshape(D, H)
    return jnp.concatenate([A, B], axis=1)


def _head_bcast_mat():
    return jnp.repeat(jnp.eye(H, dtype=jnp.float32), F, axis=1)


def _head_mean_mat():
    return jnp.tile(jnp.eye(F, dtype=jnp.float32), (H, 1)) / H


def kernel(x, edge_index, W1, al1, ar1, b1, W2, al2, ar2, b2):
    src = edge_index[0]
    dst = edge_index[1]
    AB1 = _attn_mat(al1, ar1)
    BA1 = _attn_mat(ar1, al1)
    AB2 = _attn_mat(al2, ar2)
    BA2 = _attn_mat(ar2, al2)
    R = _head_bcast_mat()
    M = _head_mean_mat()
    b1r = b1.reshape(1, D)
    b2r = b2.reshape(1, D)

    h1, elr1, erl1 = _tc_pre(x, W1, AB1, BA1)
    acc1 = _sc_edges(h1, elr1, erl1, src, dst)
    h2, elr2, erl2 = _tc_post1(acc1[0], acc1[1], b1r, R, W2, AB2, BA2)
    acc2 = _sc_edges(h2, elr2, erl2, src, dst)
    return _tc_post2(acc2[0], acc2[1], b2r, R, M)


# speedup vs baseline: 1.0008x; 1.0008x over previous
import functools

import jax
import jax.numpy as jnp
from jax import lax
from jax.experimental import pallas as pl
from jax.experimental.pallas import tpu as pltpu
from jax.experimental.pallas import tpu_sc as plsc

N = 10000
E = 320000
D = 128
H = 8
F = 16
L = 16
DM = D + L
NC = 2
NS = 16
NW = NC * NS
EPW = E // NW
K = 80
NCHUNK = EPW // K
NBLK = N // K

BN = 1000


def _tc_pre_body(x_ref, w_ref, ab_ref, ba_ref, h_ref, elr_ref, erl_ref):
    h = jnp.dot(x_ref[...], w_ref[...], preferred_element_type=jnp.float32)
    h_ref[...] = jnp.concatenate([h, jnp.zeros((BN, L), jnp.float32)], axis=1)
    elr_ref[...] = jnp.dot(h, ab_ref[...], preferred_element_type=jnp.float32)
    erl_ref[...] = jnp.dot(h, ba_ref[...], preferred_element_type=jnp.float32)


def _tc_pre(x, W, AB, BA):
    return pl.pallas_call(
        _tc_pre_body,
        grid=(N // BN,),
        in_specs=[
            pl.BlockSpec((BN, D), lambda i: (i, 0)),
            pl.BlockSpec((D, D), lambda i: (0, 0)),
            pl.BlockSpec((D, 2 * H), lambda i: (0, 0)),
            pl.BlockSpec((D, 2 * H), lambda i: (0, 0)),
        ],
        out_specs=[
            pl.BlockSpec((BN, DM), lambda i: (i, 0)),
            pl.BlockSpec((BN, 2 * H), lambda i: (i, 0)),
            pl.BlockSpec((BN, 2 * H), lambda i: (i, 0)),
        ],
        out_shape=[
            jax.ShapeDtypeStruct((N, DM), jnp.float32),
            jax.ShapeDtypeStruct((N, 2 * H), jnp.float32),
            jax.ShapeDtypeStruct((N, 2 * H), jnp.float32),
        ],
    )(x, W, AB, BA)


def _norm_block(a0_ref, a1_ref, b_ref, r_ref):
    a = a0_ref[...] + a1_ref[...]
    feat = a[:, :D]
    s8 = a[:, D:D + H]
    rec = jnp.where(s8 > 0, 1.0 / s8, 0.0)
    rbig = jnp.dot(rec, r_ref[...], preferred_element_type=jnp.float32)
    return feat * rbig + b_ref[...]


def _tc_post1_body(a0_ref, a1_ref, b_ref, r_ref, w2_ref, ab2_ref, ba2_ref,
                   h2_ref, elr2_ref, erl2_ref):
    o = _norm_block(a0_ref, a1_ref, b_ref, r_ref)
    o = jnp.where(o > 0, o, jnp.exp(o) - 1.0)
    h2 = jnp.dot(o, w2_ref[...], preferred_element_type=jnp.float32)
    h2_ref[...] = jnp.concatenate([h2, jnp.zeros((BN, L), jnp.float32)], axis=1)
    elr2_ref[...] = jnp.dot(h2, ab2_ref[...], preferred_element_type=jnp.float32)
    erl2_ref[...] = jnp.dot(h2, ba2_ref[...], preferred_element_type=jnp.float32)


def _tc_post1(a0, a1, b, R, W2, AB2, BA2):
    return pl.pallas_call(
        _tc_post1_body,
        grid=(N // BN,),
        in_specs=[
            pl.BlockSpec((BN, DM), lambda i: (i, 0)),
            pl.BlockSpec((BN, DM), lambda i: (i, 0)),
            pl.BlockSpec((1, D), lambda i: (0, 0)),
            pl.BlockSpec((H, D), lambda i: (0, 0)),
            pl.BlockSpec((D, D), lambda i: (0, 0)),
            pl.BlockSpec((D, 2 * H), lambda i: (0, 0)),
            pl.BlockSpec((D, 2 * H), lambda i: (0, 0)),
        ],
        out_specs=[
            pl.BlockSpec((BN, DM), lambda i: (i, 0)),
            pl.BlockSpec((BN, 2 * H), lambda i: (i, 0)),
            pl.BlockSpec((BN, 2 * H), lambda i: (i, 0)),
        ],
        out_shape=[
            jax.ShapeDtypeStruct((N, DM), jnp.float32),
            jax.ShapeDtypeStruct((N, 2 * H), jnp.float32),
            jax.ShapeDtypeStruct((N, 2 * H), jnp.float32),
        ],
    )(a0, a1, b, R, W2, AB2, BA2)


def _tc_post2_body(a0_ref, a1_ref, b_ref, r_ref, m_ref, out_ref):
    o = _norm_block(a0_ref, a1_ref, b_ref, r_ref)
    out_ref[...] = jnp.dot(o, m_ref[...], preferred_element_type=jnp.float32)


def _tc_post2(a0, a1, b, R, M):
    return pl.pallas_call(
        _tc_post2_body,
        grid=(N // BN,),
        in_specs=[
            pl.BlockSpec((BN, DM), lambda i: (i, 0)),
            pl.BlockSpec((BN, DM), lambda i: (i, 0)),
            pl.BlockSpec((1, D), lambda i: (0, 0)),
            pl.BlockSpec((H, D), lambda i: (0, 0)),
            pl.BlockSpec((D, F), lambda i: (0, 0)),
        ],
        out_specs=pl.BlockSpec((BN, F), lambda i: (i, 0)),
        out_shape=jax.ShapeDtypeStruct((N, F), jnp.float32),
    )(a0, a1, b, R, M)


def _sc_edge_body(h_hbm, elr_hbm, erl_hbm, src_hbm, dst_hbm, acc_out,
                  acc_sh, srcv, dstv, dsts, hrows, elsrc, erdst,
                  gsem, ssem, isem):
    c = lax.axis_index("c")
    s_ = lax.axis_index("s")
    zero16 = jnp.zeros((L,), jnp.float32)
    lane_iota = lax.iota(jnp.int32, L)

    def _zero_buf(i, _):
        for jj in range(DM // L):
            hrows[0, i, pl.ds(jj * L, L)] = zero16
        return 0
    lax.fori_loop(0, K, _zero_buf, 0)

    def _zero_acc(b, _):
        @pl.when((b % NS) == s_)
        def _():
            pltpu.sync_copy(hrows.at[0],
                            acc_sh.at[pl.ds(pl.multiple_of(b * K, 8), K)])
        return 0
    lax.fori_loop(0, NBLK, _zero_acc, 0)

    plsc.subcore_barrier()

    ebase = (c * NS + s_) * EPW

    def _fire_idx(j, par):
        base = pl.multiple_of(ebase + j * K, 8)
        pltpu.async_copy(src_hbm.at[pl.ds(base, K)], srcv.at[par], isem)
        pltpu.async_copy(dst_hbm.at[pl.ds(base, K)], dstv.at[par], isem)

    def _wait_idx(par):
        pltpu.make_async_copy(src_hbm.at[pl.ds(0, K)], srcv.at[par], isem).wait()
        pltpu.make_async_copy(dst_hbm.at[pl.ds(0, K)], dstv.at[par], isem).wait()

    def _fire_gathers(ring, par):
        pltpu.async_copy(h_hbm.at[srcv.at[par]], hrows.at[ring], gsem)
        pltpu.async_copy(elr_hbm.at[srcv.at[par]], elsrc.at[par], gsem)
        pltpu.async_copy(erl_hbm.at[dstv.at[par]], erdst.at[par], gsem)

    def _drain_gathers():
        pltpu.make_async_copy(h_hbm.at[srcv.at[0]], hrows.at[0], gsem).wait()
        pltpu.make_async_copy(elr_hbm.at[srcv.at[0]], elsrc.at[0], gsem).wait()
        pltpu.make_async_copy(erl_hbm.at[dstv.at[0]], erdst.at[0], gsem).wait()

    def _drain_scatter():
        pltpu.make_async_copy(hrows.at[0], acc_sh.at[dsts.at[0]], ssem).wait()

    def _compute(ring, par):
        def _edge(kk, _):
            for u in range(4):
                k = kk * 4 + u
                e16 = elsrc[par, k, :] + erdst[par, k, :]
                e16 = jnp.where(e16 > 0.0, e16, 0.2 * e16)
                ee = jnp.where(lane_iota < H, jnp.exp(e16), 0.0)
                hrows[ring, k, pl.ds(D, L)] = ee
                for hh in range(H):
                    hrows[ring, k, pl.ds(hh * L, L)] = (
                        hrows[ring, k, pl.ds(hh * L, L)] * ee[hh])
            return 0
        lax.fori_loop(0, K // 4, _edge, 0)

    def _fire_scatter(ring, par):
        for g in range(K // L):
            dsts[par, pl.ds(g * L, L)] = dstv[par, pl.ds(g * L, L)]
        pltpu.async_copy(hrows.at[ring], acc_sh.at[dsts.at[par]], ssem,
                         add=True)

    pltpu.sync_copy(src_hbm.at[pl.ds(pl.multiple_of(ebase, 8), K)], srcv.at[0])
    pltpu.sync_copy(dst_hbm.at[pl.ds(pl.multiple_of(ebase, 8), K)], dstv.at[0])
    _fire_gathers(0, 0)
    _fire_idx(1, 1)

    def _iter(j, _):
        p2 = lax.rem(j, 2)
        p3 = lax.rem(j, 3)
        _drain_gathers()

        @pl.when(j >= 2)
        def _():
            _drain_scatter()

        @pl.when(j <= NCHUNK - 2)
        def _():
            _wait_idx(1 - p2)
            _fire_gathers(lax.rem(j + 1, 3), 1 - p2)

        _compute(p3, p2)
        _fire_scatter(p3, p2)

        @pl.when(j <= NCHUNK - 3)
        def _():
            _fire_idx(j + 2, p2)
        return 0

    lax.fori_loop(0, NCHUNK, _iter, 0)
    _drain_scatter()
    _drain_scatter()

    plsc.subcore_barrier()

    def _wb(b, _):
        @pl.when((b % NS) == s_)
        def _():
            r = pl.multiple_of(b * K, 8)
            pltpu.sync_copy(acc_sh.at[pl.ds(r, K)], hrows.at[0])
            pltpu.sync_copy(hrows.at[0], acc_out.at[c, pl.ds(r, K)])
        return 0
    lax.fori_loop(0, NBLK, _wb, 0)


@functools.lru_cache(maxsize=1)
def _sc_edges_fn():
    return pl.kernel(
        _sc_edge_body,
        out_type=jax.ShapeDtypeStruct((NC, N, DM), jnp.float32),
        mesh=plsc.VectorSubcoreMesh(core_axis_name="c", subcore_axis_name="s",
                                    num_cores=NC, num_subcores=NS),
        compiler_params=pltpu.CompilerParams(use_tc_tiling_on_sc=False),
        scratch_types=[
            pltpu.VMEM_SHARED((N, DM), jnp.float32),
            pltpu.VMEM((2, K), jnp.int32),
            pltpu.VMEM((2, K), jnp.int32),
            pltpu.VMEM((2, K), jnp.int32),
            pltpu.VMEM((3, K, DM), jnp.float32),
            pltpu.VMEM((2, K, L), jnp.float32),
            pltpu.VMEM((2, K, L), jnp.float32),
            pltpu.SemaphoreType.DMA,
            pltpu.SemaphoreType.DMA,
            pltpu.SemaphoreType.DMA,
        ],
    )


def _sc_edges(h, elr, erl, src, dst):
    return _sc_edges_fn()(h, elr, erl, src, dst)


def _attn_mat(al, ar):
    eye = jnp.eye(H, dtype=jnp.float32)
    A = (eye[:, None, :] * al[:, :, None]).reshape(D, H)
    B = (eye[:, None, :] * ar[:, :, None]).reshape(D, H)
    return jnp.concatenate([A, B], axis=1)


def _head_bcast_mat():
    return jnp.repeat(jnp.eye(H, dtype=jnp.float32), F, axis=1)


def _head_mean_mat():
    return jnp.tile(jnp.eye(F, dtype=jnp.float32), (H, 1)) / H


def kernel(x, edge_index, W1, al1, ar1, b1, W2, al2, ar2, b2):
    src = edge_index[0]
    dst = edge_index[1]
    AB1 = _attn_mat(al1, ar1)
    BA1 = _attn_mat(ar1, al1)
    AB2 = _attn_mat(al2, ar2)
    BA2 = _attn_mat(ar2, al2)
    R = _head_bcast_mat()
    M = _head_mean_mat()
    b1r = b1.reshape(1, D)
    b2r = b2.reshape(1, D)

    h1, elr1, erl1 = _tc_pre(x, W1, AB1, BA1)
    acc1 = _sc_edges(h1, elr1, erl1, src, dst)
    h2, elr2, erl2 = _tc_post1(acc1[0], acc1[1], b1r, R, W2, AB2, BA2)
    acc2 = _sc_edges(h2, elr2, erl2, src, dst)
    return _tc_post2(acc2[0], acc2[1], b2r, R, M)


# speedup vs baseline: 1.0104x; 1.0096x over previous
import functools

import jax
import jax.numpy as jnp
from jax import lax
from jax.experimental import pallas as pl
from jax.experimental.pallas import tpu as pltpu
from jax.experimental.pallas import tpu_sc as plsc

N = 10000
E = 320000
D = 128
H = 8
F = 16
L = 16
DM = D + L
NC = 2
NS = 16
NW = NC * NS
EPW = E // NW
K = 80
NCHUNK = EPW // K
NBLK = N // K

BN = 1000


def _tc_pre_body(x_ref, w_ref, ab_ref, ba_ref, h_ref, erl_ref):
    h = jnp.dot(x_ref[...], w_ref[...], preferred_element_type=jnp.float32)
    elr = jnp.dot(h, ab_ref[...], preferred_element_type=jnp.float32)
    h_ref[...] = jnp.concatenate([h, elr], axis=1)
    erl_ref[...] = jnp.dot(h, ba_ref[...], preferred_element_type=jnp.float32)


def _tc_pre(x, W, AB, BA):
    return pl.pallas_call(
        _tc_pre_body,
        grid=(N // BN,),
        in_specs=[
            pl.BlockSpec((BN, D), lambda i: (i, 0)),
            pl.BlockSpec((D, D), lambda i: (0, 0)),
            pl.BlockSpec((D, 2 * H), lambda i: (0, 0)),
            pl.BlockSpec((D, 2 * H), lambda i: (0, 0)),
        ],
        out_specs=[
            pl.BlockSpec((BN, DM), lambda i: (i, 0)),
            pl.BlockSpec((BN, 2 * H), lambda i: (i, 0)),
        ],
        out_shape=[
            jax.ShapeDtypeStruct((N, DM), jnp.float32),
            jax.ShapeDtypeStruct((N, 2 * H), jnp.float32),
        ],
    )(x, W, AB, BA)


def _norm_block(a0_ref, a1_ref, b_ref, r_ref):
    a = a0_ref[...] + a1_ref[...]
    feat = a[:, :D]
    s8 = a[:, D:D + H]
    rec = jnp.where(s8 > 0, 1.0 / s8, 0.0)
    rbig = jnp.dot(rec, r_ref[...], preferred_element_type=jnp.float32)
    return feat * rbig + b_ref[...]


def _tc_post1_body(a0_ref, a1_ref, b_ref, r_ref, w2_ref, ab2_ref, ba2_ref,
                   h2_ref, erl2_ref):
    o = _norm_block(a0_ref, a1_ref, b_ref, r_ref)
    o = jnp.where(o > 0, o, jnp.exp(o) - 1.0)
    h2 = jnp.dot(o, w2_ref[...], preferred_element_type=jnp.float32)
    elr2 = jnp.dot(h2, ab2_ref[...], preferred_element_type=jnp.float32)
    h2_ref[...] = jnp.concatenate([h2, elr2], axis=1)
    erl2_ref[...] = jnp.dot(h2, ba2_ref[...], preferred_element_type=jnp.float32)


def _tc_post1(a0, a1, b, R, W2, AB2, BA2):
    return pl.pallas_call(
        _tc_post1_body,
        grid=(N // BN,),
        in_specs=[
            pl.BlockSpec((BN, DM), lambda i: (i, 0)),
            pl.BlockSpec((BN, DM), lambda i: (i, 0)),
            pl.BlockSpec((1, D), lambda i: (0, 0)),
            pl.BlockSpec((H, D), lambda i: (0, 0)),
            pl.BlockSpec((D, D), lambda i: (0, 0)),
            pl.BlockSpec((D, 2 * H), lambda i: (0, 0)),
            pl.BlockSpec((D, 2 * H), lambda i: (0, 0)),
        ],
        out_specs=[
            pl.BlockSpec((BN, DM), lambda i: (i, 0)),
            pl.BlockSpec((BN, 2 * H), lambda i: (i, 0)),
        ],
        out_shape=[
            jax.ShapeDtypeStruct((N, DM), jnp.float32),
            jax.ShapeDtypeStruct((N, 2 * H), jnp.float32),
        ],
    )(a0, a1, b, R, W2, AB2, BA2)


def _tc_post2_body(a0_ref, a1_ref, b_ref, r_ref, m_ref, out_ref):
    o = _norm_block(a0_ref, a1_ref, b_ref, r_ref)
    out_ref[...] = jnp.dot(o, m_ref[...], preferred_element_type=jnp.float32)


def _tc_post2(a0, a1, b, R, M):
    return pl.pallas_call(
        _tc_post2_body,
        grid=(N // BN,),
        in_specs=[
            pl.BlockSpec((BN, DM), lambda i: (i, 0)),
            pl.BlockSpec((BN, DM), lambda i: (i, 0)),
            pl.BlockSpec((1, D), lambda i: (0, 0)),
            pl.BlockSpec((H, D), lambda i: (0, 0)),
            pl.BlockSpec((D, F), lambda i: (0, 0)),
        ],
        out_specs=pl.BlockSpec((BN, F), lambda i: (i, 0)),
        out_shape=jax.ShapeDtypeStruct((N, F), jnp.float32),
    )(a0, a1, b, R, M)


def _sc_edge_body(h_hbm, erl_hbm, src_hbm, dst_hbm, acc_out,
                  acc_sh, srcv, dstv, dsts, hrows, erdst,
                  gsem, ssem, isem):
    c = lax.axis_index("c")
    s_ = lax.axis_index("s")
    zero16 = jnp.zeros((L,), jnp.float32)
    lane_iota = lax.iota(jnp.int32, L)

    def _zero_buf(i, _):
        for jj in range(DM // L):
            hrows[0, i, pl.ds(jj * L, L)] = zero16
        return 0
    lax.fori_loop(0, K, _zero_buf, 0)

    def _zero_acc(b, _):
        @pl.when((b % NS) == s_)
        def _():
            pltpu.sync_copy(hrows.at[0],
                            acc_sh.at[pl.ds(pl.multiple_of(b * K, 8), K)])
        return 0
    lax.fori_loop(0, NBLK, _zero_acc, 0)

    plsc.subcore_barrier()

    ebase = (c * NS + s_) * EPW

    def _fire_idx(j, par):
        base = pl.multiple_of(ebase + j * K, 8)
        pltpu.async_copy(src_hbm.at[pl.ds(base, K)], srcv.at[par], isem)
        pltpu.async_copy(dst_hbm.at[pl.ds(base, K)], dstv.at[par], isem)

    def _wait_idx(par):
        pltpu.make_async_copy(src_hbm.at[pl.ds(0, K)], srcv.at[par], isem).wait()
        pltpu.make_async_copy(dst_hbm.at[pl.ds(0, K)], dstv.at[par], isem).wait()

    def _fire_gathers(ring, par):
        pltpu.async_copy(h_hbm.at[srcv.at[par]], hrows.at[ring], gsem)
        pltpu.async_copy(erl_hbm.at[dstv.at[par]], erdst.at[par], gsem)

    def _drain_gathers():
        pltpu.make_async_copy(h_hbm.at[srcv.at[0]], hrows.at[0], gsem).wait()
        pltpu.make_async_copy(erl_hbm.at[dstv.at[0]], erdst.at[0], gsem).wait()

    def _drain_scatter():
        pltpu.make_async_copy(hrows.at[0], acc_sh.at[dsts.at[0]], ssem).wait()

    def _compute(ring, par):
        def _edge(kk, _):
            for u in range(4):
                k = kk * 4 + u
                e16 = hrows[ring, k, pl.ds(D, L)] + erdst[par, k, :]
                e16 = jnp.where(e16 > 0.0, e16, 0.2 * e16)
                ee = jnp.where(lane_iota < H, jnp.exp(e16), 0.0)
                hrows[ring, k, pl.ds(D, L)] = ee
                for hh in range(H):
                    hrows[ring, k, pl.ds(hh * L, L)] = (
                        hrows[ring, k, pl.ds(hh * L, L)] * ee[hh])
            return 0
        lax.fori_loop(0, K // 4, _edge, 0)

    def _fire_scatter(ring, par):
        for g in range(K // L):
            dsts[par, pl.ds(g * L, L)] = dstv[par, pl.ds(g * L, L)]
        pltpu.async_copy(hrows.at[ring], acc_sh.at[dsts.at[par]], ssem,
                         add=True)

    pltpu.sync_copy(src_hbm.at[pl.ds(pl.multiple_of(ebase, 8), K)], srcv.at[0])
    pltpu.sync_copy(dst_hbm.at[pl.ds(pl.multiple_of(ebase, 8), K)], dstv.at[0])
    _fire_gathers(0, 0)
    _fire_idx(1, 1)

    def _iter(j, _):
        p2 = lax.rem(j, 2)
        p3 = lax.rem(j, 3)
        _drain_gathers()

        @pl.when(j >= 2)
        def _():
            _drain_scatter()

        @pl.when(j <= NCHUNK - 2)
        def _():
            _wait_idx(1 - p2)
            _fire_gathers(lax.rem(j + 1, 3), 1 - p2)

        _compute(p3, p2)
        _fire_scatter(p3, p2)

        @pl.when(j <= NCHUNK - 3)
        def _():
            _fire_idx(j + 2, p2)
        return 0

    lax.fori_loop(0, NCHUNK, _iter, 0)
    _drain_scatter()
    _drain_scatter()

    plsc.subcore_barrier()

    def _wb(b, _):
        @pl.when((b % NS) == s_)
        def _():
            r = pl.multiple_of(b * K, 8)
            pltpu.sync_copy(acc_sh.at[pl.ds(r, K)], hrows.at[0])
            pltpu.sync_copy(hrows.at[0], acc_out.at[c, pl.ds(r, K)])
        return 0
    lax.fori_loop(0, NBLK, _wb, 0)


@functools.lru_cache(maxsize=1)
def _sc_edges_fn():
    return pl.kernel(
        _sc_edge_body,
        out_type=jax.ShapeDtypeStruct((NC, N, DM), jnp.float32),
        mesh=plsc.VectorSubcoreMesh(core_axis_name="c", subcore_axis_name="s",
                                    num_cores=NC, num_subcores=NS),
        compiler_params=pltpu.CompilerParams(use_tc_tiling_on_sc=False),
        scratch_types=[
            pltpu.VMEM_SHARED((N, DM), jnp.float32),
            pltpu.VMEM((2, K), jnp.int32),
            pltpu.VMEM((2, K), jnp.int32),
            pltpu.VMEM((2, K), jnp.int32),
            pltpu.VMEM((3, K, DM), jnp.float32),
            pltpu.VMEM((2, K, L), jnp.float32),
            pltpu.SemaphoreType.DMA,
            pltpu.SemaphoreType.DMA,
            pltpu.SemaphoreType.DMA,
        ],
    )


def _sc_edges(h, erl, src, dst):
    return _sc_edges_fn()(h, erl, src, dst)


def _attn_mat(al, ar):
    eye = jnp.eye(H, dtype=jnp.float32)
    A = (eye[:, None, :] * al[:, :, None]).reshape(D, H)
    B = (eye[:, None, :] * ar[:, :, None]).reshape(D, H)
    return jnp.concatenate([A, B], axis=1)


def _head_bcast_mat():
    return jnp.repeat(jnp.eye(H, dtype=jnp.float32), F, axis=1)


def _head_mean_mat():
    return jnp.tile(jnp.eye(F, dtype=jnp.float32), (H, 1)) / H


def kernel(x, edge_index, W1, al1, ar1, b1, W2, al2, ar2, b2):
    src = edge_index[0]
    dst = edge_index[1]
    AB1 = _attn_mat(al1, ar1)
    BA1 = _attn_mat(ar1, al1)
    AB2 = _attn_mat(al2, ar2)
    BA2 = _attn_mat(ar2, al2)
    R = _head_bcast_mat()
    M = _head_mean_mat()
    b1r = b1.reshape(1, D)
    b2r = b2.reshape(1, D)

    h1, erl1 = _tc_pre(x, W1, AB1, BA1)
    acc1 = _sc_edges(h1, erl1, src, dst)
    h2, erl2 = _tc_post1(acc1[0], acc1[1], b1r, R, W2, AB2, BA2)
    acc2 = _sc_edges(h2, erl2, src, dst)
    return _tc_post2(acc2[0], acc2[1], b2r, R, M)
